# Initial kernel scaffold; baseline (speedup 1.0000x reference)
import jax
import jax.numpy as jnp
from jax.experimental import pallas as pl


def kernel(mem, idx, val, sample_idx):
    raise NotImplementedError("write your pallas kernel here")


# speedup vs baseline: 3.7315x; 3.7315x over previous
import jax
import jax.numpy as jnp
from jax import lax
from jax.experimental import pallas as pl
from jax.experimental.pallas import tpu as pltpu
from jax.experimental.pallas import tpu_sc as plsc

CAP = 1_000_000
DIM = 32
N_ADD = 16384
N_SAMPLE = 4096
L = 16
NW = 32
RANGE = CAP // NW
TBL = RANGE + (-RANGE) % L
DUMMY = N_SAMPLE


def _body(mem_h, idx_h, val_h, smp_h, out_h,
          idx_v, smp_v, tbl_v, bad_v, mj_v, mo_v, us_v, uo_v, rows_v, sem):
    cid = lax.axis_index("c")
    sid = lax.axis_index("s")
    wid = sid * 2 + cid
    lo = wid * RANGE
    iota = lax.iota(jnp.int32, L)

    pltpu.sync_copy(idx_h, idx_v)
    pltpu.sync_copy(smp_h, smp_v)

    neg1 = jnp.full((L,), -1, jnp.int32)

    def init_body(i, c):
        tbl_v[pl.ds(i * L, L)] = neg1
        return c

    lax.fori_loop(0, TBL // L, init_body, 0)

    zero = jnp.zeros((L,), jnp.int32)
    dummy = jnp.full((L,), DUMMY, jnp.int32)

    def pre_body(i, c):
        s = pl.ds(i * L, L)
        mj_v[s] = zero
        mo_v[s] = dummy
        us_v[s] = zero
        uo_v[s] = dummy
        return c

    lax.fori_loop(0, N_SAMPLE // L, pre_body, 0)

    def scat_body(i, nbad):
        v = idx_v[pl.ds(i * L, L)]
        m = (v >= lo) & (v < lo + RANGE)
        t = jnp.where(m, v - lo, 0)
        j = i * L + iota
        plsc.store_scatter(tbl_v, [t], j, mask=m)
        g = plsc.load_gather(tbl_v, [t])
        flag = m & (g < j)
        fi = jnp.where(flag, 1, 0).astype(jnp.int32)
        pos = nbad + plsc.cumsum(fi) - 1
        posc = jnp.where(flag, pos, 0)
        packed = t * (N_ADD) + j
        plsc.store_scatter(bad_v, [posc], packed, mask=flag)
        return nbad + jnp.sum(fi)

    nbad = lax.fori_loop(0, N_ADD // L, scat_body, jnp.int32(0))

    def fix_body(i, c):
        chunk = lax.shift_right_logical(i, 4)
        lane = lax.bitwise_and(i, L - 1)
        pk = bad_v[pl.ds(chunk * L, L)]
        vp = lax.shift_right_logical(pk, 14)
        vj = lax.bitwise_and(pk, N_ADD - 1)
        onemask = iota == lane
        g = plsc.load_gather(tbl_v, [vp])
        plsc.store_scatter(tbl_v, [vp], jnp.maximum(g, vj), mask=onemask)
        return c

    lax.fori_loop(0, nbad, fix_body, 0)

    def smp_body(i, offs):
        om, ou = offs
        sv = smp_v[pl.ds(i * L, L)]
        m = (sv >= lo) & (sv < lo + RANGE)
        t = jnp.where(m, sv - lo, 0)
        g = plsc.load_gather(tbl_v, [t])
        mm = m & (g >= 0)
        mu = m & (g < 0)
        spos = i * L + iota
        im = jnp.where(mm, 1, 0).astype(jnp.int32)
        iu = jnp.where(mu, 1, 0).astype(jnp.int32)
        pm = om + plsc.cumsum(im) - 1
        pu = ou + plsc.cumsum(iu) - 1
        pmc = jnp.where(mm, pm, 0)
        puc = jnp.where(mu, pu, 0)
        plsc.store_scatter(mj_v, [pmc], g, mask=mm)
        plsc.store_scatter(mo_v, [pmc], spos, mask=mm)
        plsc.store_scatter(us_v, [puc], sv, mask=mu)
        plsc.store_scatter(uo_v, [puc], spos, mask=mu)
        return (om + jnp.sum(im), ou + jnp.sum(iu))

    nm, nu = lax.fori_loop(0, N_SAMPLE // L, smp_body,
                           (jnp.int32(0), jnp.int32(0)))

    def dma_m(c, carry):
        src = mj_v[pl.ds(c * L, L)]
        dst = mo_v[pl.ds(c * L, L)]
        pltpu.async_copy(val_h.at[src], rows_v, sem).wait()
        pltpu.async_copy(rows_v, out_h.at[dst], sem).wait()
        return carry

    lax.fori_loop(0, (nm + L - 1) // L, dma_m, 0)

    def dma_u(c, carry):
        src = us_v[pl.ds(c * L, L)]
        dst = uo_v[pl.ds(c * L, L)]
        pltpu.async_copy(mem_h.at[src], rows_v, sem).wait()
        pltpu.async_copy(rows_v, out_h.at[dst], sem).wait()
        return carry

    lax.fori_loop(0, (nu + L - 1) // L, dma_u, 0)


_sc_call_cache = []


def _get_sc_call():
    if not _sc_call_cache:
        _sc_call_cache.append(_build_sc_call())
    return _sc_call_cache[0]


def _build_sc_call():
    return pl.kernel(
        _body,
        out_type=jax.ShapeDtypeStruct((N_SAMPLE + 8, DIM), jnp.float32),
        mesh=plsc.VectorSubcoreMesh(core_axis_name="c", subcore_axis_name="s"),
        compiler_params=pltpu.CompilerParams(needs_layout_passes=False,
                                             use_tc_tiling_on_sc=False),
        scratch_types=[
            pltpu.VMEM((N_ADD,), jnp.int32),
            pltpu.VMEM((N_SAMPLE,), jnp.int32),
            pltpu.VMEM((TBL,), jnp.int32),
            pltpu.VMEM((N_ADD,), jnp.int32),
            pltpu.VMEM((N_SAMPLE,), jnp.int32),
            pltpu.VMEM((N_SAMPLE,), jnp.int32),
            pltpu.VMEM((N_SAMPLE,), jnp.int32),
            pltpu.VMEM((N_SAMPLE,), jnp.int32),
            pltpu.VMEM((L, DIM), jnp.float32),
            pltpu.SemaphoreType.DMA,
        ],
    )


def kernel(mem, idx, val, sample_idx):
    out = _get_sc_call()(mem, idx, val, sample_idx)
    return out[:N_SAMPLE]
